# Initial kernel scaffold; baseline (speedup 1.0000x reference)
import jax
import jax.numpy as jnp
from jax.experimental import pallas as pl


def kernel(x_global_features, h, pos_pxpypz_at_vertex, chi_squared_tracks, batch_idx):
    raise NotImplementedError("write your pallas kernel here")


# speedup vs baseline: 8.5748x; 8.5748x over previous
import functools

import jax
import jax.numpy as jnp
from jax import lax
from jax.experimental import pallas as pl
from jax.experimental.pallas import tpu as pltpu
from jax.experimental.pallas import tpu_sc as plsc

NN = 100000
NSEG = 1000
OB = 1008
NC, NS = 2, 16
NW = NC * NS
CH = 3200
NV = CH // 16
SPW = 32
INF = float("inf")

_mesh = plsc.VectorSubcoreMesh(
    core_axis_name="c", subcore_axis_name="s", num_cores=NC, num_subcores=NS)


def _wid():
    return lax.axis_index("s") * NC + lax.axis_index("c")


def _iota16():
    return lax.broadcasted_iota(jnp.int32, (16,), 0)


def _bc(x):
    return jnp.broadcast_to(x, (16,))


@functools.partial(
    pl.kernel,
    out_type=(
        jax.ShapeDtypeStruct((NW, OB), jnp.float32),
        jax.ShapeDtypeStruct((NW, OB), jnp.int32),
    ),
    mesh=_mesh,
    compiler_params=pltpu.CompilerParams(
        use_tc_tiling_on_sc=False, needs_layout_passes=False),
    scratch_types=[
        pltpu.VMEM((CH, 16), jnp.float32),
        pltpu.VMEM((CH,), jnp.float32),
        pltpu.VMEM((CH,), jnp.int32),
        pltpu.VMEM((OB,), jnp.float32),
        pltpu.VMEM((OB,), jnp.int32),
    ],
)
def _phase1(h_hbm, chi_hbm, bidx_hbm, pval_hbm, pidx_hbm,
            h16_v, chi_v, bidx_v, oval_v, oidx_v):
    wid = _wid()
    base = jnp.minimum(wid * CH, NN - CH)
    lanes = _iota16()

    pltpu.sync_copy(chi_hbm.at[pl.ds(base, CH)], chi_v)
    pltpu.sync_copy(bidx_hbm.at[pl.ds(base, CH)], bidx_v)
    pltpu.sync_copy(h_hbm.at[pl.ds(base, CH), pl.ds(0, 16)], h16_v)

    inf_vec = jnp.full((16,), INF, jnp.float32)
    big_idx = jnp.full((16,), NN, jnp.int32)

    def init(i, _):
        oval_v[pl.ds(i * 16, 16)] = inf_vec
        oidx_v[pl.ds(i * 16, 16)] = big_idx
        return 0

    lax.fori_loop(0, OB // 16, init, 0)

    lane0 = lanes == 0

    def step(j, _):
        off = j * 16
        vb = bidx_v[pl.ds(off, 16)]
        vc = chi_v[pl.ds(off, 16)]
        rows = _bc(off) + lanes
        h3 = plsc.load_gather(h16_v, [rows, _bc(jnp.int32(3))])
        h4 = plsc.load_gather(h16_v, [rows, _bc(jnp.int32(4))])
        h5 = plsc.load_gather(h16_v, [rows, _bc(jnp.int32(5))])
        h6 = plsc.load_gather(h16_v, [rows, _bc(jnp.int32(6))])
        filt = (h4 > h3) & (h4 >= h5) & (h4 >= h6)
        key = jnp.where(filt, vc, INF)

        def cond(rem):
            return jnp.any(rem != 0)

        def body(rem):
            remb = rem != 0
            s = jnp.min(jnp.where(remb, vb, jnp.int32(2147483647)))
            svec = _bc(s)
            segm = vb == svec
            kseg = jnp.where(segm, key, INF)
            m = jnp.min(kseg)
            mvec = _bc(m)
            eq = segm & (kseg == mvec)
            lane = plsc.all_reduce_ffs(eq)
            node_win = _bc(base + off) + _bc(lane).astype(jnp.int32)
            cur = plsc.load_gather(oval_v, [svec])
            better = (mvec < cur) & lane0
            plsc.store_scatter(oval_v, [svec], mvec, mask=better)
            plsc.store_scatter(oidx_v, [svec], node_win, mask=better)
            return jnp.where(segm, 0, rem)

        lax.while_loop(cond, body, jnp.ones((16,), jnp.int32))
        return 0

    lax.fori_loop(0, NV, step, 0)

    pltpu.sync_copy(oval_v, pval_hbm.at[wid])
    pltpu.sync_copy(oidx_v, pidx_hbm.at[wid])


@functools.partial(
    pl.kernel,
    out_type=(
        jax.ShapeDtypeStruct((NSEG,), jnp.float32),
        jax.ShapeDtypeStruct((NSEG * 3,), jnp.float32),
    ),
    mesh=_mesh,
    compiler_params=pltpu.CompilerParams(
        use_tc_tiling_on_sc=False, needs_layout_passes=False),
    scratch_types=[
        pltpu.VMEM((NW, SPW), jnp.float32),
        pltpu.VMEM((NW, SPW), jnp.int32),
        pltpu.VMEM((SPW,), jnp.int32),
        pltpu.VMEM((SPW * 3,), jnp.int32),
        pltpu.VMEM((SPW * 3,), jnp.float32),
        pltpu.VMEM((SPW,), jnp.float32),
        pltpu.SemaphoreType.DMA,
    ],
)
def _phase2(pval_hbm, pidx_hbm, posf_hbm, ptr_hbm, pdir_hbm,
            pv_v, pi_v, picks_v, gidx_v, pbuf_v, ptr_v, sem):
    wid = _wid()
    seg0 = jnp.minimum(wid * SPW, NSEG - SPW)
    lanes = _iota16()
    lane0 = lanes == 0

    pltpu.sync_copy(pval_hbm.at[:, pl.ds(seg0, SPW)], pv_v)
    pltpu.sync_copy(pidx_hbm.at[:, pl.ds(seg0, SPW)], pi_v)

    def pick_step(sl, _):
        colv = _bc(sl).astype(jnp.int32)
        a = plsc.load_gather(pv_v, [lanes, colv])
        b = plsc.load_gather(pv_v, [lanes + 16, colv])
        m = jnp.minimum(jnp.min(a), jnp.min(b))
        mvec = _bc(m)
        eqa = a == mvec
        anya = jnp.any(eqa)
        la = plsc.all_reduce_ffs(eqa)
        lb = plsc.all_reduce_ffs(b == mvec)
        tile = jnp.where(anya, _bc(la), _bc(lb) + 16).astype(jnp.int32)
        pk = plsc.load_gather(pi_v, [tile, colv])
        pickf = jnp.where(mvec < INF, pk, 0)
        plsc.store_scatter(picks_v, [colv], pickf, mask=lane0)
        return 0

    lax.fori_loop(0, SPW, pick_step, 0)

    def gidx_step(v, _):
        k = _bc(v * 16) + lanes
        p = k // 3
        c = k - p * 3
        pickp = plsc.load_gather(picks_v, [p])
        gidx_v[pl.ds(v * 16, 16)] = pickp * 3 + c
        return 0

    lax.fori_loop(0, (SPW * 3) // 16, gidx_step, 0)

    pltpu.async_copy(posf_hbm.at[gidx_v], pbuf_v, sem).wait()

    def norm_step(hh, _):
        b3 = (_bc(hh * 16) + lanes) * 3
        x = plsc.load_gather(pbuf_v, [b3])
        y = plsc.load_gather(pbuf_v, [b3 + 1])
        z = plsc.load_gather(pbuf_v, [b3 + 2])
        s = x * x + y * y + z * z
        i = plsc.bitcast(s, jnp.int32)
        i = jnp.int32(0x1FBD1DF5) + (i >> 1)
        r = plsc.bitcast(i, jnp.float32)
        r = 0.5 * (r + s / r)
        r = 0.5 * (r + s / r)
        r = 0.5 * (r + s / r)
        r = jnp.where(s > 0.0, r, 0.0)
        ptr_v[pl.ds(hh * 16, 16)] = r
        return 0

    lax.fori_loop(0, SPW // 16, norm_step, 0)

    pltpu.sync_copy(ptr_v, ptr_hbm.at[pl.ds(seg0, SPW)])
    pltpu.sync_copy(pbuf_v, pdir_hbm.at[pl.ds(seg0 * 3, SPW * 3)])


def kernel(x_global_features, h, pos_pxpypz_at_vertex, chi_squared_tracks, batch_idx):
    del x_global_features
    posf = jnp.reshape(pos_pxpypz_at_vertex, (-1,))
    pval, pidx = _phase1(h, chi_squared_tracks, batch_idx.astype(jnp.int32))
    p_tracks, pdir_flat = _phase2(pval, pidx, posf)
    return p_tracks, jnp.reshape(pdir_flat, (NSEG, 3))
